# baseline (speedup 1.0000x reference)
import functools

import jax
import jax.numpy as jnp
from jax import lax
from jax.experimental import pallas as pl
from jax.experimental.pallas import tpu as pltpu
from jax.experimental.pallas import tpu_sc as plsc


def kernel(U, V, pos_u_idxs, pos_v_idxs, neg_v_idxs):
    B = pos_u_idxs.shape[0]
    K = neg_v_idxs.shape[1]
    D = U.shape[1]
    L = 16
    NC, NS = 2, 16
    NW = NC * NS
    BPW = B // NW
    C = 32
    NCHUNK = BPW // C
    KP = 24
    NJ = D // L

    neg_flat = neg_v_idxs.reshape(B * K)

    mesh = plsc.VectorSubcoreMesh(
        core_axis_name="c", subcore_axis_name="s",
        num_cores=NC, num_subcores=NS)

    @functools.partial(
        pl.kernel,
        out_type=jax.ShapeDtypeStruct((NW, KP * BPW), jnp.float32),
        mesh=mesh,
        scratch_types=[
            pltpu.VMEM((C,), jnp.int32),
            pltpu.VMEM((C,), jnp.int32),
            pltpu.VMEM((C * K,), jnp.int32),
            pltpu.VMEM((C, D), jnp.float32),
            pltpu.VMEM((C, D), jnp.float32),
            pltpu.VMEM((C * K, D), jnp.float32),
            pltpu.VMEM((KP * BPW,), jnp.float32),
            pltpu.SemaphoreType.DMA,
        ],
        compiler_params=pltpu.CompilerParams(
            needs_layout_passes=False, use_tc_tiling_on_sc=False),
    )
    def sc_scores(u_hbm, v_hbm, pu_hbm, pv_hbm, nv_hbm, out_hbm,
                  idxu, idxpv, idxnv, rows_u, rows_pv, rows_nv, scores, sem):
        wid = lax.axis_index("s") * NC + lax.axis_index("c")
        gbase = wid * BPW
        lane = lax.iota(jnp.int32, L)
        last = lane == (L - 1)

        def put(score_row, col, acc, negate):
            tot = plsc.cumsum(acc)
            if negate:
                tot = -tot
            addr = jnp.broadcast_to(score_row * BPW + col, (L,)).astype(jnp.int32)
            plsc.store_scatter(scores, [addr], tot, mask=last)

        def chunk_body(ci, carry):
            ebase = gbase + ci * C
            pltpu.sync_copy(pu_hbm.at[pl.ds(ebase, C)], idxu)
            pltpu.sync_copy(pv_hbm.at[pl.ds(ebase, C)], idxpv)
            pltpu.sync_copy(nv_hbm.at[pl.ds(ebase * K, C * K)], idxnv)
            cu = pltpu.async_copy(u_hbm.at[idxu], rows_u, sem)
            cpv = pltpu.async_copy(v_hbm.at[idxpv], rows_pv, sem)
            cnv = pltpu.async_copy(v_hbm.at[idxnv], rows_nv, sem)
            cu.wait()
            cpv.wait()
            cnv.wait()

            def elem_body(e, ecarry):
                col = ci * C + e
                uvecs = [rows_u[e, pl.ds(L * j, L)] for j in range(NJ)]
                acc = uvecs[0] * rows_pv[e, pl.ds(0, L)]
                for j in range(1, NJ):
                    acc = acc + uvecs[j] * rows_pv[e, pl.ds(L * j, L)]
                put(0, col, acc, negate=False)
                for k in range(K):
                    r = e * K + k
                    acc = uvecs[0] * rows_nv[r, pl.ds(0, L)]
                    for j in range(1, NJ):
                        acc = acc + uvecs[j] * rows_nv[r, pl.ds(L * j, L)]
                    put(1 + k, col, acc, negate=True)
                return ecarry

            return lax.fori_loop(0, C, elem_body, carry)

        lax.fori_loop(0, NCHUNK, chunk_body, 0)
        pltpu.sync_copy(scores, out_hbm.at[wid])

    scores3 = sc_scores(U, V, pos_u_idxs, pos_v_idxs, neg_flat)
    scores2 = scores3.reshape(NW * KP, BPW)

    def tc_body(s_ref, o_ref):
        x = s_ref[...]
        row = lax.broadcasted_iota(jnp.int32, x.shape, 0)
        valid = (row % KP) < (1 + K)
        ls = jnp.where(valid, jnp.log(jax.nn.sigmoid(x)), 0.0)
        o_ref[0, 0] = -jnp.sum(ls) / B

    loss = pl.pallas_call(
        tc_body,
        out_shape=jax.ShapeDtypeStruct((1, 1), jnp.float32),
        out_specs=pl.BlockSpec(memory_space=pltpu.SMEM),
    )(scores2)
    return loss[0, 0]
